# baseline (speedup 1.0000x reference)
import functools

import jax
import jax.numpy as jnp
from jax import lax
from jax.experimental import pallas as pl
from jax.experimental.pallas import tpu as pltpu
from jax.experimental.pallas import tpu_sc as plsc

NN = 10000
EE = 320000
DIN = 128
DHID = 256
NCLS = 40
DL3 = 48
BN_EPS = 1e-5

NCORE = 2
NSUB = 16
CHUNK = 128
N_PAD = 10240
E_PAD = 327680
TRASH = NN
R = 1024
NB = N_PAD // R


def _zero_rows(rows_ref, nrows, width):
    z = jnp.zeros((16,), jnp.float32)

    def body(r, _):
        for k in range(width // 16):
            rows_ref[r, pl.ds(k * 16, 16)] = z
        return 0

    lax.fori_loop(0, nrows, body, 0)


def _make_agg(depth, col_split, nbuf, group):
    ept = E_PAD // NSUB if col_split else E_PAD // (NCORE * NSUB)
    n_chunks = ept // CHUNK
    core_stride = E_PAD if col_split else E_PAD // NCORE
    rows_per_tile = N_PAD // NSUB
    wb = rows_per_tile // CHUNK
    n_groups = n_chunks // group

    def body(srcf, dstf, table, out, *scr):
        isg, idg = scr[0], scr[1]
        bufs = scr[2:2 + nbuf]
        acc = scr[2 + nbuf]
        gsems = scr[3 + nbuf:3 + 2 * nbuf]
        ssems = scr[3 + 2 * nbuf:3 + 3 * nbuf]
        c = lax.axis_index("c")
        s = lax.axis_index("s")
        _zero_rows(bufs[0], CHUNK, depth)
        for j in range(wb):
            pltpu.sync_copy(bufs[0], acc.at[pl.ds(s * rows_per_tile + j * CHUNK, CHUNK)])
        plsc.subcore_barrier()

        base0 = c * core_stride + s * ept
        baseq = c * (core_stride // CHUNK) + s * n_chunks

        def ggather(k, buf, sem):
            pltpu.async_copy(table.at[isg.at[pl.ds(k * CHUNK, CHUNK)]], buf, sem)

        def gwait(buf, sem):
            pltpu.make_async_copy(table.at[isg.at[pl.ds(0, CHUNK)]], buf, sem).wait()

        def swait(buf, sem):
            pltpu.make_async_copy(buf, acc.at[idg.at[0]], sem).wait()

        def group_body(g, _):
            pltpu.sync_copy(srcf.at[pl.ds(base0 + g * (group * CHUNK), group * CHUNK)], isg)
            pltpu.sync_copy(dstf.at[pl.ds(baseq + g * group, group)], idg)
            for i in range(nbuf):
                ggather(i, bufs[i], gsems[i])

            def rot(q, _):
                k0 = q * nbuf
                for i in range(nbuf):
                    gwait(bufs[i], gsems[i])
                    pltpu.async_copy(bufs[i], acc.at[idg.at[k0 + i]], ssems[i], add=True)
                for i in range(nbuf):
                    @pl.when(k0 + i + nbuf < group)
                    def _():
                        swait(bufs[i], ssems[i])
                        ggather(k0 + i + nbuf, bufs[i], gsems[i])
                return 0

            lax.fori_loop(0, group // nbuf, rot, 0)
            for i in range(nbuf):
                swait(bufs[i], ssems[i])
            return 0

        lax.fori_loop(0, n_groups, group_body, 0)
        plsc.subcore_barrier()
        for j in range(wb):
            r0 = s * rows_per_tile + j * CHUNK
            pltpu.sync_copy(acc.at[pl.ds(r0, CHUNK)], bufs[0])
            pltpu.sync_copy(bufs[0], out.at[pl.ds(c * N_PAD + r0, CHUNK)])

    return pl.kernel(
        body,
        out_type=jax.ShapeDtypeStruct((NCORE * N_PAD, depth), jnp.float32),
        mesh=plsc.VectorSubcoreMesh(core_axis_name="c", subcore_axis_name="s", num_cores=NCORE, num_subcores=NSUB),
        scratch_types=(
            [
                pltpu.VMEM((group * CHUNK,), jnp.int32),
                pltpu.VMEM((group, CHUNK), jnp.int32),
            ]
            + [pltpu.VMEM((CHUNK, depth), jnp.float32)] * nbuf
            + [pltpu.MemorySpace.VMEM_SHARED((N_PAD, depth), jnp.float32)]
            + [pltpu.SemaphoreType.DMA] * (2 * nbuf)
        ),
        compiler_params=pltpu.CompilerParams(use_tc_tiling_on_sc=False),
        name=f"gcn_agg_d{depth}_{'col' if col_split else 'edge'}",
    )


def _make_degrees():
    ept = E_PAD // (NCORE * NSUB)
    n_chunks = ept // CHUNK
    rows_per_tile = N_PAD // NSUB
    wb = rows_per_tile // CHUNK

    group = 16
    n_groups = n_chunks // group

    def body(srcf, dstf, out_s, out_d, isg, idg, ones, tmp, hist_s, hist_d,
             sem_s, sem_d):
        c = lax.axis_index("c")
        s = lax.axis_index("s")
        _zero_rows(tmp, CHUNK, 16)
        one = jnp.ones((16,), jnp.float32)

        def fill(r, _):
            ones[r, pl.ds(0, 16)] = one
            return 0

        lax.fori_loop(0, CHUNK, fill, 0)
        for j in range(wb):
            pltpu.sync_copy(tmp, hist_s.at[pl.ds(s * rows_per_tile + j * CHUNK, CHUNK)])
            pltpu.sync_copy(tmp, hist_d.at[pl.ds(s * rows_per_tile + j * CHUNK, CHUNK)])
        plsc.subcore_barrier()

        baseq = (c * (E_PAD // NCORE) + s * ept) // CHUNK

        def group_body(g, _):
            pltpu.sync_copy(srcf.at[pl.ds(baseq + g * group, group)], isg)
            pltpu.sync_copy(dstf.at[pl.ds(baseq + g * group, group)], idg)

            def fire(k, _):
                pltpu.async_copy(ones, hist_s.at[isg.at[k]], sem_s, add=True)
                pltpu.async_copy(ones, hist_d.at[idg.at[k]], sem_d, add=True)
                return 0

            lax.fori_loop(0, group, fire, 0)

            def drain(k, _):
                pltpu.make_async_copy(ones, hist_s.at[isg.at[0]], sem_s).wait()
                pltpu.make_async_copy(ones, hist_d.at[idg.at[0]], sem_d).wait()
                return 0

            lax.fori_loop(0, group, drain, 0)
            return 0

        lax.fori_loop(0, n_groups, group_body, 0)
        plsc.subcore_barrier()
        for j in range(wb):
            r0 = s * rows_per_tile + j * CHUNK
            pltpu.sync_copy(hist_s.at[pl.ds(r0, CHUNK)], tmp)
            pltpu.sync_copy(tmp, out_s.at[pl.ds(c * N_PAD + r0, CHUNK)])
            pltpu.sync_copy(hist_d.at[pl.ds(r0, CHUNK)], tmp)
            pltpu.sync_copy(tmp, out_d.at[pl.ds(c * N_PAD + r0, CHUNK)])

    return pl.kernel(
        body,
        out_type=[
            jax.ShapeDtypeStruct((NCORE * N_PAD, 16), jnp.float32),
            jax.ShapeDtypeStruct((NCORE * N_PAD, 16), jnp.float32),
        ],
        mesh=plsc.VectorSubcoreMesh(core_axis_name="c", subcore_axis_name="s", num_cores=NCORE, num_subcores=NSUB),
        scratch_types=[
            pltpu.VMEM((16, CHUNK), jnp.int32),
            pltpu.VMEM((16, CHUNK), jnp.int32),
            pltpu.VMEM((CHUNK, 16), jnp.float32),
            pltpu.VMEM((CHUNK, 16), jnp.float32),
            pltpu.MemorySpace.VMEM_SHARED((N_PAD, 16), jnp.float32),
            pltpu.MemorySpace.VMEM_SHARED((N_PAD, 16), jnp.float32),
            pltpu.SemaphoreType.DMA,
            pltpu.SemaphoreType.DMA,
        ],
        compiler_params=pltpu.CompilerParams(use_tc_tiling_on_sc=False),
        name="gcn_degrees",
    )


_BN_RS = float(1.0 / (1.0 + BN_EPS) ** 0.5)


def _prep_body(x, hs0, hs1, hd0, hd1, xs, ns, nd):
    deg_o = jnp.maximum(hs0[:, 0:1] + hs1[:, 0:1], 1.0)
    deg_i = jnp.maximum(hd0[:, 0:1] + hd1[:, 0:1], 1.0)
    n_s = lax.rsqrt(deg_o)
    n_d = lax.rsqrt(deg_i)
    xs[...] = x[...] * n_s
    ns[...] = jnp.broadcast_to(n_s, (R, 8))
    nd[...] = jnp.broadcast_to(n_d, (R, 8))


def _tc_prep(x_pad, hist_s, hist_d):
    hs_spec0 = pl.BlockSpec((R, 16), lambda i: (i, 0))
    hs_spec1 = pl.BlockSpec((R, 16), lambda i: (NB + i, 0))
    return pl.pallas_call(
        _prep_body,
        grid=(NB,),
        in_specs=[
            pl.BlockSpec((R, DIN), lambda i: (i, 0)),
            hs_spec0, hs_spec1, hs_spec0, hs_spec1,
        ],
        out_specs=[
            pl.BlockSpec((R, DIN), lambda i: (i, 0)),
            pl.BlockSpec((R, 8), lambda i: (i, 0)),
            pl.BlockSpec((R, 8), lambda i: (i, 0)),
        ],
        out_shape=[
            jax.ShapeDtypeStruct((N_PAD, DIN), jnp.float32),
            jax.ShapeDtypeStruct((N_PAD, 8), jnp.float32),
            jax.ShapeDtypeStruct((N_PAD, 8), jnp.float32),
        ],
    )(x_pad, hist_s, hist_s, hist_d, hist_d)


def _l1_body(p0, p1, w, b, g, bt, nd, ns, out):
    a = p0[...] + p1[...]
    h = jnp.dot(a, w[...], preferred_element_type=jnp.float32)
    h = h * nd[:, 0:1] + b[...]
    h = (h * _BN_RS) * g[...] + bt[...]
    h = jnp.maximum(h, 0.0)
    out[...] = h * ns[:, 0:1]


def _tc_layer1(agg_x, w1, b1, g1, bt1, nd, ns):
    nspec = pl.BlockSpec((R, 8), lambda c, i: (i, 0))
    pspec = pl.BlockSpec((1, DHID // 2), lambda c, i: (0, c))
    return pl.pallas_call(
        _l1_body,
        grid=(NCORE, NB),
        in_specs=[
            pl.BlockSpec((R, DIN), lambda c, i: (i, 0)),
            pl.BlockSpec((R, DIN), lambda c, i: (NB + i, 0)),
            pl.BlockSpec((DIN, DHID // 2), lambda c, i: (0, c)),
            pspec, pspec, pspec, nspec, nspec,
        ],
        out_specs=pl.BlockSpec((R, DHID // 2), lambda c, i: (c * NB + i, 0)),
        out_shape=jax.ShapeDtypeStruct((NCORE * N_PAD, DHID // 2), jnp.float32),
    )(agg_x, agg_x, w1, b1, g1, bt1, nd, ns)


def _l2_body(a0, a1, w2a, w2b, b, g, bt, w3, nd, ns, out):
    h = jnp.dot(a0[...], w2a[...], preferred_element_type=jnp.float32)
    h += jnp.dot(a1[...], w2b[...], preferred_element_type=jnp.float32)
    h = h * nd[:, 0:1] + b[...]
    h = (h * _BN_RS) * g[...] + bt[...]
    h = jnp.maximum(h, 0.0)
    h = h * ns[:, 0:1]
    out[...] = jnp.dot(h, w3[...], preferred_element_type=jnp.float32)


def _tc_layer2(agg2, w2, b2, g2, bt2, w3p, nd, ns):
    nspec = pl.BlockSpec((R, 8), lambda i: (i, 0))
    pspec = pl.BlockSpec((1, DHID), lambda i: (0, 0))
    half = DHID // 2
    return pl.pallas_call(
        _l2_body,
        grid=(NB,),
        in_specs=[
            pl.BlockSpec((R, half), lambda i: (i, 0)),
            pl.BlockSpec((R, half), lambda i: (NB + i, 0)),
            pl.BlockSpec((half, DHID), lambda i: (0, 0)),
            pl.BlockSpec((half, DHID), lambda i: (0, 0)),
            pspec, pspec, pspec,
            pl.BlockSpec((DHID, DL3), lambda i: (0, 0)),
            nspec, nspec,
        ],
        out_specs=pl.BlockSpec((R, DL3), lambda i: (i, 0)),
        out_shape=jax.ShapeDtypeStruct((N_PAD, DL3), jnp.float32),
    )(agg2, agg2, w2.reshape(2, half, DHID)[0], w2.reshape(2, half, DHID)[1],
      b2, g2, bt2, w3p, nd, ns)


def _final_body(p0, p1, b, nd, out):
    a = (p0[...] + p1[...]) * nd[:, 0:1] + b[...]
    t = a[:, :NCLS]
    m = jnp.max(t, axis=1, keepdims=True)
    e = jnp.exp(t - m)
    se = jnp.sum(e, axis=1, keepdims=True)
    out[...] = (t - m) - jnp.log(se)


def _tc_final(agg3, b3p, nd):
    return pl.pallas_call(
        _final_body,
        grid=(NB,),
        in_specs=[
            pl.BlockSpec((R, DL3), lambda i: (i, 0)),
            pl.BlockSpec((R, DL3), lambda i: (NB + i, 0)),
            pl.BlockSpec((1, DL3), lambda i: (0, 0)),
            pl.BlockSpec((R, 8), lambda i: (i, 0)),
        ],
        out_specs=pl.BlockSpec((R, NCLS), lambda i: (i, 0)),
        out_shape=jax.ShapeDtypeStruct((N_PAD, NCLS), jnp.float32),
    )(agg3, agg3, b3p, nd)


@jax.jit
def kernel(x, edge_index, W1, b1, gamma1, beta1, W2, b2, gamma2, beta2, W3, b3):
    f32 = jnp.float32
    pad_e = E_PAD - EE
    pad_idx = TRASH + jnp.arange(pad_e, dtype=jnp.int32) % (N_PAD - NN)
    src = jnp.concatenate([edge_index[0], pad_idx])
    dst = jnp.concatenate([edge_index[1], pad_idx])
    src2 = jnp.concatenate([src, src + N_PAD])
    dst2 = jnp.concatenate([dst, dst])
    x_pad = jnp.zeros((N_PAD, DIN), f32).at[:NN].set(x)

    dst3 = dst.reshape(-1, CHUNK)
    src3 = src.reshape(-1, CHUNK)
    hist_s, hist_d = _make_degrees()(src3, dst3)
    xs, ns, nd = _tc_prep(x_pad, hist_s, hist_d)

    dst23 = dst2.reshape(-1, CHUNK)
    agg_x = _make_agg(DIN, col_split=False, nbuf=2, group=40)(src, dst3, xs)
    h1s = _tc_layer1(agg_x, W1, b1.reshape(1, -1), gamma1.reshape(1, -1),
                     beta1.reshape(1, -1), nd, ns)

    agg2 = _make_agg(DHID // 2, col_split=True, nbuf=2, group=40)(src2, dst23, h1s)
    w3p = jnp.zeros((DHID, DL3), f32).at[:, :NCLS].set(W3)
    y3 = _tc_layer2(agg2, W2, b2.reshape(1, -1), gamma2.reshape(1, -1),
                    beta2.reshape(1, -1), w3p, nd, ns)

    agg3 = _make_agg(DL3, col_split=False, nbuf=8, group=80)(src, dst3, y3)
    b3p = jnp.zeros((1, DL3), f32).at[0, :NCLS].set(b3)
    out = _tc_final(agg3, b3p, nd)
    return out[:NN]


# speedup vs baseline: 1.2156x; 1.2156x over previous
import functools

import jax
import jax.numpy as jnp
from jax import lax
from jax.experimental import pallas as pl
from jax.experimental.pallas import tpu as pltpu
from jax.experimental.pallas import tpu_sc as plsc

NN = 10000
EE = 320000
DIN = 128
DHID = 256
NCLS = 40
DL3 = 48
BN_EPS = 1e-5

NCORE = 2
NSUB = 16
CHUNK = 128
N_PAD = 10240
E_PAD = 327680
TRASH = NN
R = 1024
NB = N_PAD // R


def _zero_rows(rows_ref, nrows, width):
    z = jnp.zeros((16,), jnp.float32)

    def body(r, _):
        for k in range(width // 16):
            rows_ref[r, pl.ds(k * 16, 16)] = z
        return 0

    lax.fori_loop(0, nrows, body, 0)


def _make_agg(depth, col_split, nbuf, group):
    ept = E_PAD // NSUB if col_split else E_PAD // (NCORE * NSUB)
    n_chunks = ept // CHUNK
    core_stride = E_PAD if col_split else E_PAD // NCORE
    rows_per_tile = N_PAD // NSUB
    wb = rows_per_tile // CHUNK
    n_groups = n_chunks // group

    def body(srcf, dstf, table, out, *scr):
        isg, idg = scr[0], scr[1]
        bufs = scr[2:2 + nbuf]
        acc = scr[2 + nbuf]
        gsems = scr[3 + nbuf:3 + 2 * nbuf]
        ssems = scr[3 + 2 * nbuf:3 + 3 * nbuf]
        c = lax.axis_index("c")
        s = lax.axis_index("s")
        _zero_rows(bufs[0], CHUNK, depth)
        for j in range(wb):
            pltpu.sync_copy(bufs[0], acc.at[pl.ds(s * rows_per_tile + j * CHUNK, CHUNK)])
        plsc.subcore_barrier()

        base0 = c * core_stride + s * ept
        baseq = c * (core_stride // CHUNK) + s * n_chunks

        def ggather(k, buf, sem):
            pltpu.async_copy(table.at[isg.at[pl.ds(k * CHUNK, CHUNK)]], buf, sem)

        def gwait(buf, sem):
            pltpu.make_async_copy(table.at[isg.at[pl.ds(0, CHUNK)]], buf, sem).wait()

        def swait(buf, sem):
            pltpu.make_async_copy(buf, acc.at[idg.at[0]], sem).wait()

        def group_body(g, _):
            pltpu.sync_copy(srcf.at[pl.ds(base0 + g * (group * CHUNK), group * CHUNK)], isg)
            pltpu.sync_copy(dstf.at[pl.ds(baseq + g * group, group)], idg)
            for i in range(nbuf):
                ggather(i, bufs[i], gsems[i])

            if nbuf == 2:
                def pair(p, _):
                    k0 = 2 * p
                    gwait(bufs[0], gsems[0])
                    pltpu.sync_copy(bufs[0], acc.at[idg.at[k0]], add=True)

                    @pl.when(k0 + 2 < group)
                    def _():
                        ggather(k0 + 2, bufs[0], gsems[0])

                    gwait(bufs[1], gsems[1])
                    pltpu.sync_copy(bufs[1], acc.at[idg.at[k0 + 1]], add=True)

                    @pl.when(k0 + 3 < group)
                    def _():
                        ggather(k0 + 3, bufs[1], gsems[1])
                    return 0

                lax.fori_loop(0, group // 2, pair, 0)
                return 0

            def rot(q, _):
                k0 = q * nbuf
                for i in range(nbuf):
                    gwait(bufs[i], gsems[i])
                    pltpu.async_copy(bufs[i], acc.at[idg.at[k0 + i]], ssems[i], add=True)
                for i in range(nbuf):
                    @pl.when(k0 + i + nbuf < group)
                    def _():
                        swait(bufs[i], ssems[i])
                        ggather(k0 + i + nbuf, bufs[i], gsems[i])
                return 0

            lax.fori_loop(0, group // nbuf, rot, 0)
            for i in range(nbuf):
                swait(bufs[i], ssems[i])
            return 0

        lax.fori_loop(0, n_groups, group_body, 0)
        plsc.subcore_barrier()
        for j in range(wb):
            r0 = s * rows_per_tile + j * CHUNK
            pltpu.sync_copy(acc.at[pl.ds(r0, CHUNK)], bufs[0])
            pltpu.sync_copy(bufs[0], out.at[pl.ds(c * N_PAD + r0, CHUNK)])

    return pl.kernel(
        body,
        out_type=jax.ShapeDtypeStruct((NCORE * N_PAD, depth), jnp.float32),
        mesh=plsc.VectorSubcoreMesh(core_axis_name="c", subcore_axis_name="s", num_cores=NCORE, num_subcores=NSUB),
        scratch_types=(
            [
                pltpu.VMEM((group * CHUNK,), jnp.int32),
                pltpu.VMEM((group, CHUNK), jnp.int32),
            ]
            + [pltpu.VMEM((CHUNK, depth), jnp.float32)] * nbuf
            + [pltpu.MemorySpace.VMEM_SHARED((N_PAD, depth), jnp.float32)]
            + [pltpu.SemaphoreType.DMA] * (2 * nbuf)
        ),
        compiler_params=pltpu.CompilerParams(use_tc_tiling_on_sc=False),
        name=f"gcn_agg_d{depth}_{'col' if col_split else 'edge'}",
    )


def _make_degrees():
    ept = E_PAD // (NCORE * NSUB)
    n_chunks = ept // CHUNK
    rows_per_tile = N_PAD // NSUB
    wb = rows_per_tile // CHUNK

    group = 16
    n_groups = n_chunks // group

    def body(srcf, dstf, out_s, out_d, isg, idg, ones, tmp, hist_s, hist_d,
             sem_s, sem_d):
        c = lax.axis_index("c")
        s = lax.axis_index("s")
        _zero_rows(tmp, CHUNK, 16)
        one = jnp.ones((16,), jnp.float32)

        def fill(r, _):
            ones[r, pl.ds(0, 16)] = one
            return 0

        lax.fori_loop(0, CHUNK, fill, 0)
        for j in range(wb):
            pltpu.sync_copy(tmp, hist_s.at[pl.ds(s * rows_per_tile + j * CHUNK, CHUNK)])
            pltpu.sync_copy(tmp, hist_d.at[pl.ds(s * rows_per_tile + j * CHUNK, CHUNK)])
        plsc.subcore_barrier()

        baseq = (c * (E_PAD // NCORE) + s * ept) // CHUNK

        def group_body(g, _):
            pltpu.sync_copy(srcf.at[pl.ds(baseq + g * group, group)], isg)
            pltpu.sync_copy(dstf.at[pl.ds(baseq + g * group, group)], idg)

            def fire(k, _):
                pltpu.async_copy(ones, hist_s.at[isg.at[k]], sem_s, add=True)
                pltpu.async_copy(ones, hist_d.at[idg.at[k]], sem_d, add=True)
                return 0

            lax.fori_loop(0, group, fire, 0)

            def drain(k, _):
                pltpu.make_async_copy(ones, hist_s.at[isg.at[0]], sem_s).wait()
                pltpu.make_async_copy(ones, hist_d.at[idg.at[0]], sem_d).wait()
                return 0

            lax.fori_loop(0, group, drain, 0)
            return 0

        lax.fori_loop(0, n_groups, group_body, 0)
        plsc.subcore_barrier()
        for j in range(wb):
            r0 = s * rows_per_tile + j * CHUNK
            pltpu.sync_copy(hist_s.at[pl.ds(r0, CHUNK)], tmp)
            pltpu.sync_copy(tmp, out_s.at[pl.ds(c * N_PAD + r0, CHUNK)])
            pltpu.sync_copy(hist_d.at[pl.ds(r0, CHUNK)], tmp)
            pltpu.sync_copy(tmp, out_d.at[pl.ds(c * N_PAD + r0, CHUNK)])

    return pl.kernel(
        body,
        out_type=[
            jax.ShapeDtypeStruct((NCORE * N_PAD, 16), jnp.float32),
            jax.ShapeDtypeStruct((NCORE * N_PAD, 16), jnp.float32),
        ],
        mesh=plsc.VectorSubcoreMesh(core_axis_name="c", subcore_axis_name="s", num_cores=NCORE, num_subcores=NSUB),
        scratch_types=[
            pltpu.VMEM((16, CHUNK), jnp.int32),
            pltpu.VMEM((16, CHUNK), jnp.int32),
            pltpu.VMEM((CHUNK, 16), jnp.float32),
            pltpu.VMEM((CHUNK, 16), jnp.float32),
            pltpu.MemorySpace.VMEM_SHARED((N_PAD, 16), jnp.float32),
            pltpu.MemorySpace.VMEM_SHARED((N_PAD, 16), jnp.float32),
            pltpu.SemaphoreType.DMA,
            pltpu.SemaphoreType.DMA,
        ],
        compiler_params=pltpu.CompilerParams(use_tc_tiling_on_sc=False),
        name="gcn_degrees",
    )


_BN_RS = float(1.0 / (1.0 + BN_EPS) ** 0.5)


def _prep_body(x, hs0, hs1, hd0, hd1, xs, ns, nd):
    deg_o = jnp.maximum(hs0[:, 0:1] + hs1[:, 0:1], 1.0)
    deg_i = jnp.maximum(hd0[:, 0:1] + hd1[:, 0:1], 1.0)
    n_s = lax.rsqrt(deg_o)
    n_d = lax.rsqrt(deg_i)
    xs[...] = x[...] * n_s
    ns[...] = jnp.broadcast_to(n_s, (R, 8))
    nd[...] = jnp.broadcast_to(n_d, (R, 8))


def _tc_prep(x_pad, hist_s, hist_d):
    hs_spec0 = pl.BlockSpec((R, 16), lambda i: (i, 0))
    hs_spec1 = pl.BlockSpec((R, 16), lambda i: (NB + i, 0))
    return pl.pallas_call(
        _prep_body,
        grid=(NB,),
        in_specs=[
            pl.BlockSpec((R, DIN), lambda i: (i, 0)),
            hs_spec0, hs_spec1, hs_spec0, hs_spec1,
        ],
        out_specs=[
            pl.BlockSpec((R, DIN), lambda i: (i, 0)),
            pl.BlockSpec((R, 8), lambda i: (i, 0)),
            pl.BlockSpec((R, 8), lambda i: (i, 0)),
        ],
        out_shape=[
            jax.ShapeDtypeStruct((N_PAD, DIN), jnp.float32),
            jax.ShapeDtypeStruct((N_PAD, 8), jnp.float32),
            jax.ShapeDtypeStruct((N_PAD, 8), jnp.float32),
        ],
    )(x_pad, hist_s, hist_s, hist_d, hist_d)


def _l1_body(p0, p1, w, b, g, bt, nd, ns, out):
    a = p0[...] + p1[...]
    h = jnp.dot(a, w[...], preferred_element_type=jnp.float32)
    h = h * nd[:, 0:1] + b[...]
    h = (h * _BN_RS) * g[...] + bt[...]
    h = jnp.maximum(h, 0.0)
    out[...] = h * ns[:, 0:1]


def _tc_layer1(agg_x, w1, b1, g1, bt1, nd, ns):
    nspec = pl.BlockSpec((R, 8), lambda c, i: (i, 0))
    pspec = pl.BlockSpec((1, DHID // 2), lambda c, i: (0, c))
    return pl.pallas_call(
        _l1_body,
        grid=(NCORE, NB),
        in_specs=[
            pl.BlockSpec((R, DIN), lambda c, i: (i, 0)),
            pl.BlockSpec((R, DIN), lambda c, i: (NB + i, 0)),
            pl.BlockSpec((DIN, DHID // 2), lambda c, i: (0, c)),
            pspec, pspec, pspec, nspec, nspec,
        ],
        out_specs=pl.BlockSpec((R, DHID // 2), lambda c, i: (c * NB + i, 0)),
        out_shape=jax.ShapeDtypeStruct((NCORE * N_PAD, DHID // 2), jnp.float32),
    )(agg_x, agg_x, w1, b1, g1, bt1, nd, ns)


def _l2_body(a0, a1, w2a, w2b, b, g, bt, w3, nd, ns, out):
    h = jnp.dot(a0[...], w2a[...], preferred_element_type=jnp.float32)
    h += jnp.dot(a1[...], w2b[...], preferred_element_type=jnp.float32)
    h = h * nd[:, 0:1] + b[...]
    h = (h * _BN_RS) * g[...] + bt[...]
    h = jnp.maximum(h, 0.0)
    h = h * ns[:, 0:1]
    out[...] = jnp.dot(h, w3[...], preferred_element_type=jnp.float32)


def _tc_layer2(agg2, w2, b2, g2, bt2, w3p, nd, ns):
    nspec = pl.BlockSpec((R, 8), lambda i: (i, 0))
    pspec = pl.BlockSpec((1, DHID), lambda i: (0, 0))
    half = DHID // 2
    return pl.pallas_call(
        _l2_body,
        grid=(NB,),
        in_specs=[
            pl.BlockSpec((R, half), lambda i: (i, 0)),
            pl.BlockSpec((R, half), lambda i: (NB + i, 0)),
            pl.BlockSpec((half, DHID), lambda i: (0, 0)),
            pl.BlockSpec((half, DHID), lambda i: (0, 0)),
            pspec, pspec, pspec,
            pl.BlockSpec((DHID, DL3), lambda i: (0, 0)),
            nspec, nspec,
        ],
        out_specs=pl.BlockSpec((R, DL3), lambda i: (i, 0)),
        out_shape=jax.ShapeDtypeStruct((N_PAD, DL3), jnp.float32),
    )(agg2, agg2, w2.reshape(2, half, DHID)[0], w2.reshape(2, half, DHID)[1],
      b2, g2, bt2, w3p, nd, ns)


def _final_body(p0, p1, b, nd, out):
    a = (p0[...] + p1[...]) * nd[:, 0:1] + b[...]
    t = a[:, :NCLS]
    m = jnp.max(t, axis=1, keepdims=True)
    e = jnp.exp(t - m)
    se = jnp.sum(e, axis=1, keepdims=True)
    out[...] = (t - m) - jnp.log(se)


def _tc_final(agg3, b3p, nd):
    return pl.pallas_call(
        _final_body,
        grid=(NB,),
        in_specs=[
            pl.BlockSpec((R, DL3), lambda i: (i, 0)),
            pl.BlockSpec((R, DL3), lambda i: (NB + i, 0)),
            pl.BlockSpec((1, DL3), lambda i: (0, 0)),
            pl.BlockSpec((R, 8), lambda i: (i, 0)),
        ],
        out_specs=pl.BlockSpec((R, NCLS), lambda i: (i, 0)),
        out_shape=jax.ShapeDtypeStruct((N_PAD, NCLS), jnp.float32),
    )(agg3, agg3, b3p, nd)


@jax.jit
def kernel(x, edge_index, W1, b1, gamma1, beta1, W2, b2, gamma2, beta2, W3, b3):
    f32 = jnp.float32
    pad_e = E_PAD - EE
    pad_idx = TRASH + jnp.arange(pad_e, dtype=jnp.int32) % (N_PAD - NN)
    src = jnp.concatenate([edge_index[0], pad_idx])
    dst = jnp.concatenate([edge_index[1], pad_idx])
    src2 = jnp.concatenate([src, src + N_PAD])
    dst2 = jnp.concatenate([dst, dst])
    x_pad = jnp.zeros((N_PAD, DIN), f32).at[:NN].set(x)

    dst3 = dst.reshape(-1, CHUNK)
    src3 = src.reshape(-1, CHUNK)
    hist_s, hist_d = _make_degrees()(src3, dst3)
    xs, ns, nd = _tc_prep(x_pad, hist_s, hist_d)

    dst23 = dst2.reshape(-1, CHUNK)
    agg_x = _make_agg(DIN, col_split=False, nbuf=2, group=40)(src, dst3, xs)
    h1s = _tc_layer1(agg_x, W1, b1.reshape(1, -1), gamma1.reshape(1, -1),
                     beta1.reshape(1, -1), nd, ns)

    agg2 = _make_agg(DHID // 2, col_split=True, nbuf=2, group=40)(src2, dst23, h1s)
    w3p = jnp.zeros((DHID, DL3), f32).at[:, :NCLS].set(W3)
    y3 = _tc_layer2(agg2, W2, b2.reshape(1, -1), gamma2.reshape(1, -1),
                    beta2.reshape(1, -1), w3p, nd, ns)

    agg3 = _make_agg(DL3, col_split=False, nbuf=8, group=80)(src, dst3, y3)
    b3p = jnp.zeros((1, DL3), f32).at[0, :NCLS].set(b3)
    out = _tc_final(agg3, b3p, nd)
    return out[:NN]
